# baseline (speedup 1.0000x reference)
import jax
import jax.numpy as jnp
from jax.experimental import pallas as pl
from jax.experimental.pallas import tpu as pltpu

_K = 8
_B = 8


def _ring_body(x_hbm, o_hbm, buf, sem_in, sem_out):
    rows = x_hbm.shape[0]
    c = rows // _K

    def in_cp(i):
        return pltpu.make_async_copy(
            x_hbm.at[pl.ds(i * c, c), :], buf.at[i % _B], sem_in)

    def out_cp(i):
        return pltpu.make_async_copy(
            buf.at[i % _B], o_hbm.at[pl.ds(i * c, c), :], sem_out)

    for i in range(_B):
        in_cp(i).start()
    for i in range(_K):
        in_cp(i).wait()
        out_cp(i).start()
        j = i + _B
        if j < _K:
            out_cp(i).wait()
            in_cp(j).start()
    for i in range(_K - _B, _K):
        out_cp(i).wait()


def kernel(x):
    rows, cols = x.shape
    return pl.pallas_call(
        _ring_body,
        out_shape=jax.ShapeDtypeStruct(x.shape, x.dtype),
        in_specs=[pl.BlockSpec(memory_space=pl.ANY)],
        out_specs=pl.BlockSpec(memory_space=pl.ANY),
        scratch_shapes=[
            pltpu.VMEM((_B, rows // _K, cols), x.dtype),
            pltpu.SemaphoreType.DMA,
            pltpu.SemaphoreType.DMA,
        ],
    )(x)


# speedup vs baseline: 1.0225x; 1.0225x over previous
import jax
import jax.numpy as jnp
from jax.experimental import pallas as pl
from jax.experimental.pallas import tpu as pltpu

_K = 2
_B = 2


def _ring_body(x_hbm, o_hbm, buf, sem_in, sem_out):
    rows = x_hbm.shape[0]
    c = rows // _K

    def in_cp(i):
        return pltpu.make_async_copy(
            x_hbm.at[pl.ds(i * c, c), :], buf.at[i % _B], sem_in)

    def out_cp(i):
        return pltpu.make_async_copy(
            buf.at[i % _B], o_hbm.at[pl.ds(i * c, c), :], sem_out)

    for i in range(_B):
        in_cp(i).start()
    for i in range(_K):
        in_cp(i).wait()
        out_cp(i).start()
        j = i + _B
        if j < _K:
            out_cp(i).wait()
            in_cp(j).start()
    for i in range(_K - _B, _K):
        out_cp(i).wait()


def kernel(x):
    rows, cols = x.shape
    return pl.pallas_call(
        _ring_body,
        out_shape=jax.ShapeDtypeStruct(x.shape, x.dtype),
        in_specs=[pl.BlockSpec(memory_space=pl.ANY)],
        out_specs=pl.BlockSpec(memory_space=pl.ANY),
        scratch_shapes=[
            pltpu.VMEM((_B, rows // _K, cols), x.dtype),
            pltpu.SemaphoreType.DMA,
            pltpu.SemaphoreType.DMA,
        ],
    )(x)
